# baseline (speedup 1.0000x reference)
import functools

import jax
import jax.numpy as jnp
from jax import lax
from jax.experimental import pallas as pl
from jax.experimental.pallas import tpu as pltpu
from jax.experimental.pallas import tpu_sc as plsc

N_FIELDS = 26
VOCAB = 100000
DIM = 32
B = 1024
T = 50

BLK_B = 128
NBLK_B = B // BLK_B
NW = 32
BLK_W = DIM * BLK_B
N_GROUPS = 2
F_G = N_FIELDS // N_GROUPS

_mesh = plsc.VectorSubcoreMesh(core_axis_name="c", subcore_axis_name="s")


def _make_gather(n_fields):
    n_blocks = n_fields * T * NBLK_B
    per_w = (n_blocks + NW - 1) // NW
    n_pair = (per_w + 2) // 2

    @functools.partial(
        pl.kernel,
        mesh=_mesh,
        out_type=jax.ShapeDtypeStruct((n_fields * T * DIM * B,), jnp.float32),
        scratch_types=[
            pltpu.VMEM((BLK_B,), jnp.int32),
            pltpu.VMEM((BLK_B,), jnp.int32),
            pltpu.VMEM((BLK_B, DIM), jnp.float32),
            pltpu.VMEM((BLK_B, DIM), jnp.float32),
            pltpu.VMEM((BLK_W,), jnp.float32),
            pltpu.VMEM((BLK_W,), jnp.float32),
            pltpu.SemaphoreType.DMA,
            pltpu.SemaphoreType.DMA,
            pltpu.SemaphoreType.DMA,
            pltpu.SemaphoreType.DMA,
            pltpu.SemaphoreType.DMA,
            pltpu.SemaphoreType.DMA,
        ],
        compiler_params=pltpu.CompilerParams(
            use_tc_tiling_on_sc=False, needs_layout_passes=False
        ),
    )
    def sc_gather(
        idx_hbm, tab_hbm, out_hbm,
        idx_a, idx_b, gath_a, gath_b, tr_a, tr_b,
        isem_a, isem_b, gsem_a, gsem_b, osem_a, osem_b,
    ):
        cid = lax.axis_index("c")
        sid = lax.axis_index("s")
        wid = sid * 2 + cid
        base = wid * per_w
        last = jnp.minimum(base + per_w - 1, n_blocks - 1)
        base = jnp.minimum(base, last)

        idx_v = [idx_a, idx_b]
        gath_v = [gath_a, gath_b]
        tr_v = [tr_a, tr_b]
        isem = [isem_a, isem_b]
        gsem = [gsem_a, gsem_b]
        osem = [osem_a, osem_b]

        scat = [lax.iota(jnp.int32, 16) * BLK_B + h * 16 * BLK_B
                for h in range(2)]

        def do_slot(m, s, j):
            u = jnp.minimum(base + s, last)
            un = jnp.minimum(base + s + 1, last)
            up = jnp.minimum(base + s + 2, last)
            pltpu.make_async_copy(idx_hbm.at[pl.ds(0, BLK_B)], idx_v[j ^ 1],
                                  isem[j ^ 1]).wait()
            fn = un // (T * NBLK_B)
            pltpu.async_copy(tab_hbm.at[fn].at[idx_v[j ^ 1]], gath_v[j ^ 1],
                             gsem[j ^ 1])
            pltpu.make_async_copy(tab_hbm.at[0].at[pl.ds(0, BLK_B), :],
                                  gath_v[j], gsem[j]).wait()
            pltpu.async_copy(idx_hbm.at[pl.ds(up * BLK_B, BLK_B)], idx_v[j],
                             isem[j])
            @pl.when(m > 0)
            def _():
                pltpu.make_async_copy(out_hbm.at[pl.ds(0, BLK_W)], tr_v[j],
                                      osem[j]).wait()
            prev = None
            for row in range(BLK_B):
                cur = [gath_v[j][row, pl.ds(h * 16, 16)] for h in range(2)]
                if prev is not None:
                    for h in range(2):
                        plsc.store_scatter(tr_v[j], [scat[h] + (row - 1)],
                                           prev[h])
                prev = cur
            for h in range(2):
                plsc.store_scatter(tr_v[j], [scat[h] + (BLK_B - 1)], prev[h])
            ft = u // NBLK_B
            bc = u % NBLK_B
            out_base = ft * (DIM * B) + bc * (8 * BLK_B)
            for dt in range(4):
                pltpu.async_copy(
                    tr_v[j].at[pl.ds(dt * 8 * BLK_B, 8 * BLK_B)],
                    out_hbm.at[pl.ds(out_base + dt * (8 * B), 8 * BLK_B)],
                    osem[j],
                )

        def pair_body(m, carry):
            do_slot(m, 2 * m, 0)
            do_slot(m, 2 * m + 1, 1)
            return carry

        pltpu.sync_copy(idx_hbm.at[pl.ds(base * BLK_B, BLK_B)], idx_v[0])
        f0 = base // (T * NBLK_B)
        pltpu.async_copy(tab_hbm.at[f0].at[idx_v[0]], gath_v[0], gsem[0])
        pltpu.async_copy(idx_hbm.at[pl.ds(jnp.minimum(base + 1, last) * BLK_B,
                                          BLK_B)], idx_v[1], isem[1])
        lax.fori_loop(0, n_pair, pair_body, 0)
        pltpu.make_async_copy(tab_hbm.at[0].at[pl.ds(0, BLK_B), :], gath_v[0],
                              gsem[0]).wait()
        pltpu.make_async_copy(idx_hbm.at[pl.ds(0, BLK_B)], idx_v[1],
                              isem[1]).wait()
        pltpu.make_async_copy(out_hbm.at[pl.ds(0, BLK_W)], tr_v[0],
                              osem[0]).wait()
        pltpu.make_async_copy(out_hbm.at[pl.ds(0, BLK_W)], tr_v[1],
                              osem[1]).wait()

    return sc_gather


_gather_g = _make_gather(F_G)


def kernel(x, tables):
    flat_idx = x.transpose(2, 1, 0).reshape(N_FIELDS * T * B)
    rows_g = F_G * T * B
    outs = [
        _gather_g(
            lax.dynamic_slice_in_dim(flat_idx, g * rows_g, rows_g),
            lax.dynamic_slice_in_dim(tables, g * F_G, F_G),
        )
        for g in range(N_GROUPS)
    ]
    out = jnp.concatenate(outs)
    out = out.reshape(N_FIELDS, T, DIM // 8, B // BLK_B, 8, BLK_B)
    out = out.transpose(0, 3, 5, 1, 2, 4).reshape(N_FIELDS, B, T, DIM)
    return out


# speedup vs baseline: 1.2377x; 1.2377x over previous
import functools

import jax
import jax.numpy as jnp
from jax import lax
from jax.experimental import pallas as pl
from jax.experimental.pallas import tpu as pltpu
from jax.experimental.pallas import tpu_sc as plsc

N_FIELDS = 26
VOCAB = 100000
DIM = 32
B = 1024
T = 50

BLK_B = 128
NBLK_B = B // BLK_B
NW = 32
BLK_W = DIM * BLK_B
N_GROUPS = 1
F_G = N_FIELDS // N_GROUPS

_mesh = plsc.VectorSubcoreMesh(core_axis_name="c", subcore_axis_name="s")


def _make_gather(n_fields):
    n_blocks = n_fields * T * NBLK_B
    per_w = (n_blocks + NW - 1) // NW
    n_pair = (per_w + 2) // 2

    @functools.partial(
        pl.kernel,
        mesh=_mesh,
        out_type=jax.ShapeDtypeStruct((n_fields * T * DIM * B,), jnp.float32),
        scratch_types=[
            pltpu.VMEM((BLK_B,), jnp.int32),
            pltpu.VMEM((BLK_B,), jnp.int32),
            pltpu.VMEM((BLK_B, DIM), jnp.float32),
            pltpu.VMEM((BLK_B, DIM), jnp.float32),
            pltpu.VMEM((BLK_W,), jnp.float32),
            pltpu.VMEM((BLK_W,), jnp.float32),
            pltpu.SemaphoreType.DMA,
            pltpu.SemaphoreType.DMA,
            pltpu.SemaphoreType.DMA,
            pltpu.SemaphoreType.DMA,
            pltpu.SemaphoreType.DMA,
            pltpu.SemaphoreType.DMA,
        ],
        compiler_params=pltpu.CompilerParams(
            use_tc_tiling_on_sc=False, needs_layout_passes=False
        ),
    )
    def sc_gather(
        idx_hbm, tab_hbm, out_hbm,
        idx_a, idx_b, gath_a, gath_b, tr_a, tr_b,
        isem_a, isem_b, gsem_a, gsem_b, osem_a, osem_b,
    ):
        cid = lax.axis_index("c")
        sid = lax.axis_index("s")
        wid = sid * 2 + cid
        base = wid * per_w
        last = jnp.minimum(base + per_w - 1, n_blocks - 1)
        base = jnp.minimum(base, last)

        idx_v = [idx_a, idx_b]
        gath_v = [gath_a, gath_b]
        tr_v = [tr_a, tr_b]
        isem = [isem_a, isem_b]
        gsem = [gsem_a, gsem_b]
        osem = [osem_a, osem_b]

        scat = [lax.iota(jnp.int32, 16) * BLK_B + h * 16 * BLK_B
                for h in range(2)]

        def do_slot(m, s, j):
            u = jnp.minimum(base + s, last)
            un = jnp.minimum(base + s + 1, last)
            up = jnp.minimum(base + s + 2, last)
            pltpu.make_async_copy(idx_hbm.at[pl.ds(0, BLK_B)], idx_v[j ^ 1],
                                  isem[j ^ 1]).wait()
            fn = un // (T * NBLK_B)
            pltpu.async_copy(tab_hbm.at[fn].at[idx_v[j ^ 1]], gath_v[j ^ 1],
                             gsem[j ^ 1])
            pltpu.make_async_copy(tab_hbm.at[0].at[pl.ds(0, BLK_B), :],
                                  gath_v[j], gsem[j]).wait()
            pltpu.async_copy(idx_hbm.at[pl.ds(up * BLK_B, BLK_B)], idx_v[j],
                             isem[j])
            @pl.when(m > 0)
            def _():
                pltpu.make_async_copy(out_hbm.at[pl.ds(0, BLK_W)], tr_v[j],
                                      osem[j]).wait()
            prev = None
            for row in range(BLK_B):
                cur = [gath_v[j][row, pl.ds(h * 16, 16)] for h in range(2)]
                if prev is not None:
                    for h in range(2):
                        plsc.store_scatter(tr_v[j], [scat[h] + (row - 1)],
                                           prev[h])
                prev = cur
            for h in range(2):
                plsc.store_scatter(tr_v[j], [scat[h] + (BLK_B - 1)], prev[h])
            ft = u // NBLK_B
            bc = u % NBLK_B
            out_base = ft * (DIM * B) + bc * (8 * BLK_B)
            for dt in range(4):
                pltpu.async_copy(
                    tr_v[j].at[pl.ds(dt * 8 * BLK_B, 8 * BLK_B)],
                    out_hbm.at[pl.ds(out_base + dt * (8 * B), 8 * BLK_B)],
                    osem[j],
                )

        def pair_body(m, carry):
            do_slot(m, 2 * m, 0)
            do_slot(m, 2 * m + 1, 1)
            return carry

        pltpu.sync_copy(idx_hbm.at[pl.ds(base * BLK_B, BLK_B)], idx_v[0])
        f0 = base // (T * NBLK_B)
        pltpu.async_copy(tab_hbm.at[f0].at[idx_v[0]], gath_v[0], gsem[0])
        pltpu.async_copy(idx_hbm.at[pl.ds(jnp.minimum(base + 1, last) * BLK_B,
                                          BLK_B)], idx_v[1], isem[1])
        lax.fori_loop(0, n_pair, pair_body, 0)
        pltpu.make_async_copy(tab_hbm.at[0].at[pl.ds(0, BLK_B), :], gath_v[0],
                              gsem[0]).wait()
        pltpu.make_async_copy(idx_hbm.at[pl.ds(0, BLK_B)], idx_v[1],
                              isem[1]).wait()
        pltpu.make_async_copy(out_hbm.at[pl.ds(0, BLK_W)], tr_v[0],
                              osem[0]).wait()
        pltpu.make_async_copy(out_hbm.at[pl.ds(0, BLK_W)], tr_v[1],
                              osem[1]).wait()

    return sc_gather


_gather_g = _make_gather(F_G)


def kernel(x, tables):
    flat_idx = x.transpose(2, 1, 0).reshape(N_FIELDS * T * B)
    tabw = jax.lax.optimization_barrier(
        tables.reshape(N_FIELDS * VOCAB * DIM // 128, 128)
    )
    out = _gather_g(flat_idx, tabw.reshape(N_FIELDS, VOCAB, DIM))
    out = out.reshape(N_FIELDS, T, DIM // 8, B // BLK_B, 8, BLK_B)
    out = out.transpose(0, 3, 5, 1, 2, 4).reshape(N_FIELDS, B, T, DIM)
    return out
